# Initial kernel scaffold; baseline (speedup 1.0000x reference)
import jax
import jax.numpy as jnp
from jax.experimental import pallas as pl


def kernel(x, edge_index, W, b, gamma, beta):
    raise NotImplementedError("write your pallas kernel here")


# speedup vs baseline: 10.2599x; 10.2599x over previous
import functools

import jax
import jax.numpy as jnp
from jax import lax
from jax.experimental import pallas as pl
from jax.experimental.pallas import tpu as pltpu
from jax.experimental.pallas import tpu_sc as plsc

NC = 2
NS = 16
CHUNK = 128
DEG_W = 128


def _cdiv(a, b):
    return (a + b - 1) // b


def kernel(x, edge_index, W, b, gamma, beta):
    n, d = x.shape
    e = edge_index.shape[1]
    nw = NC * NS
    cpw = _cdiv(e, CHUNK * nw)
    e_pad = cpw * nw * CHUNK
    n_acc = _cdiv(n + 1, NS * CHUNK) * NS * CHUNK
    rpt = n_acc // NS
    wch = rpt // CHUNK

    row = edge_index[0]
    col = edge_index[1]
    pad = e_pad - e
    if pad:
        row = jnp.concatenate([row, jnp.full((pad,), n, jnp.int32)])
        col = jnp.concatenate([col, jnp.full((pad,), n, jnp.int32)])

    ones16 = jnp.ones((CHUNK, DEG_W), jnp.float32)
    zeros16 = jnp.zeros((CHUNK, DEG_W), jnp.float32)
    zrows = jnp.zeros((CHUNK, d), jnp.float32)

    mesh = plsc.VectorSubcoreMesh(core_axis_name="c", subcore_axis_name="s")

    @functools.partial(
        pl.kernel,
        out_type=jax.ShapeDtypeStruct((NC, n_acc, DEG_W), jnp.float32),
        mesh=mesh,
        scratch_types=[
            pltpu.VMEM((CHUNK,), jnp.int32),
            pltpu.VMEM((CHUNK, DEG_W), jnp.float32),
            pltpu.VMEM((CHUNK, DEG_W), jnp.float32),
            pltpu.VMEM_SHARED((n_acc, DEG_W), jnp.float32),
        ],
    )
    def deg_kernel(col_hbm, ones_hbm, zeros_hbm, out_hbm, cidx, vone, vzero, acc):
        cid = lax.axis_index("c")
        sid = lax.axis_index("s")
        wid = cid * NS + sid
        r0 = sid * rpt
        pltpu.sync_copy(zeros_hbm, vzero)
        pltpu.sync_copy(ones_hbm, vone)
        for k in range(wch):
            pltpu.sync_copy(vzero, acc.at[pl.ds(r0 + k * CHUNK, CHUNK)])
        plsc.subcore_barrier()

        def body(j, carry):
            base = (wid * cpw + j) * CHUNK
            pltpu.sync_copy(col_hbm.at[pl.ds(base, CHUNK)], cidx)
            pltpu.sync_copy(vone, acc.at[cidx], add=True)
            return carry

        lax.fori_loop(0, cpw, body, 0)
        plsc.subcore_barrier()
        for k in range(wch):
            pltpu.sync_copy(acc.at[pl.ds(r0 + k * CHUNK, CHUNK)], vzero)
            pltpu.sync_copy(vzero, out_hbm.at[cid, pl.ds(r0 + k * CHUNK, CHUNK)])

    degp = deg_kernel(col, ones16, zeros16)

    BM = 2000

    def mm_body(x_ref, w_ref, o_ref):
        o_ref[...] = jnp.dot(x_ref[...], w_ref[...],
                             preferred_element_type=jnp.float32)

    xw = pl.pallas_call(
        mm_body,
        grid=(n // BM,),
        in_specs=[pl.BlockSpec((BM, d), lambda i: (i, 0)),
                  pl.BlockSpec((d, d), lambda i: (0, 0))],
        out_specs=pl.BlockSpec((BM, d), lambda i: (i, 0)),
        out_shape=jax.ShapeDtypeStruct((n, d), jnp.float32),
    )(x, W)

    dp0 = degp[0, :n, 0:1]
    dp1 = degp[1, :n, 0:1]

    def s_body(xw_ref, d0_ref, d1_ref, y_ref, dv_ref):
        deg = d0_ref[...] + d1_ref[...]
        dinv = jnp.where(deg > 0.0,
                         lax.rsqrt(jnp.maximum(deg, 1e-12)), 0.0)
        y_ref[...] = xw_ref[...] * dinv
        dv_ref[...] = dinv

    y, dinv = pl.pallas_call(
        s_body,
        grid=(n // BM,),
        in_specs=[pl.BlockSpec((BM, d), lambda i: (i, 0)),
                  pl.BlockSpec((BM, 1), lambda i: (i, 0)),
                  pl.BlockSpec((BM, 1), lambda i: (i, 0))],
        out_specs=[pl.BlockSpec((BM, d), lambda i: (i, 0)),
                   pl.BlockSpec((BM, 1), lambda i: (i, 0))],
        out_shape=[jax.ShapeDtypeStruct((n, d), jnp.float32),
                   jax.ShapeDtypeStruct((n, 1), jnp.float32)],
    )(xw, dp0, dp1)

    y_pad = jnp.concatenate([y, jnp.zeros((8, d), jnp.float32)], axis=0)

    @functools.partial(
        pl.kernel,
        out_type=jax.ShapeDtypeStruct((NC, n_acc, d), jnp.float32),
        mesh=mesh,
        scratch_types=[
            pltpu.VMEM((CHUNK,), jnp.int32),
            pltpu.VMEM((CHUNK,), jnp.int32),
            pltpu.VMEM((CHUNK, d), jnp.float32),
            pltpu.VMEM_SHARED((n_acc, d), jnp.float32),
            pltpu.SemaphoreType.DMA,
        ],
    )
    def agg_kernel(y_hbm, row_hbm, col_hbm, z_hbm, out_hbm,
                   ridx, cidx, rows, acc, sem):
        cid = lax.axis_index("c")
        sid = lax.axis_index("s")
        wid = cid * NS + sid
        r0 = sid * rpt
        pltpu.sync_copy(z_hbm, rows)
        for k in range(wch):
            pltpu.sync_copy(rows, acc.at[pl.ds(r0 + k * CHUNK, CHUNK)])
        plsc.subcore_barrier()

        def body(j, carry):
            base = (wid * cpw + j) * CHUNK
            pltpu.sync_copy(row_hbm.at[pl.ds(base, CHUNK)], ridx)
            pltpu.async_copy(y_hbm.at[ridx], rows, sem).wait()
            pltpu.sync_copy(col_hbm.at[pl.ds(base, CHUNK)], cidx)
            pltpu.sync_copy(rows, acc.at[cidx], add=True)
            return carry

        lax.fori_loop(0, cpw, body, 0)
        plsc.subcore_barrier()
        for k in range(wch):
            pltpu.sync_copy(acc.at[pl.ds(r0 + k * CHUNK, CHUNK)], rows)
            pltpu.sync_copy(rows, out_hbm.at[cid, pl.ds(r0 + k * CHUNK, CHUNK)])

    aggp = agg_kernel(y_pad, row, col, zrows)

    a0 = aggp[0, :n]
    a1 = aggp[1, :n]
    b2 = b.reshape(1, d)
    g2 = gamma.reshape(1, d)
    be2 = beta.reshape(1, d)

    def f1_body(a0_ref, a1_ref, dv_ref, b_ref, q_ref, st_ref):
        i = pl.program_id(0)
        q = (a0_ref[...] + a1_ref[...]) * dv_ref[...] + b_ref[...]
        q_ref[...] = q

        @pl.when(i == 0)
        def _():
            st_ref[...] = jnp.zeros_like(st_ref)

        st_ref[0:1, :] += jnp.sum(q, axis=0, keepdims=True)
        st_ref[1:2, :] += jnp.sum(q * q, axis=0, keepdims=True)

    q, stats = pl.pallas_call(
        f1_body,
        grid=(n // BM,),
        in_specs=[pl.BlockSpec((BM, d), lambda i: (i, 0)),
                  pl.BlockSpec((BM, d), lambda i: (i, 0)),
                  pl.BlockSpec((BM, 1), lambda i: (i, 0)),
                  pl.BlockSpec((1, d), lambda i: (0, 0))],
        out_specs=[pl.BlockSpec((BM, d), lambda i: (i, 0)),
                   pl.BlockSpec((2, d), lambda i: (0, 0))],
        out_shape=[jax.ShapeDtypeStruct((n, d), jnp.float32),
                   jax.ShapeDtypeStruct((2, d), jnp.float32)],
    )(a0, a1, dinv, b2)

    def f2_body(q_ref, st_ref, g_ref, be_ref, x_ref, o_ref):
        mean = st_ref[0:1, :] * (1.0 / n)
        var = st_ref[1:2, :] * (1.0 / n) - mean * mean
        hh = (g_ref[...] * (q_ref[...] - mean) * lax.rsqrt(var + 1e-5)
              + be_ref[...])
        o_ref[...] = jnp.maximum(hh, 0.0) + x_ref[...]

    h = pl.pallas_call(
        f2_body,
        grid=(n // BM,),
        in_specs=[pl.BlockSpec((BM, d), lambda i: (i, 0)),
                  pl.BlockSpec((2, d), lambda i: (0, 0)),
                  pl.BlockSpec((1, d), lambda i: (0, 0)),
                  pl.BlockSpec((1, d), lambda i: (0, 0)),
                  pl.BlockSpec((BM, d), lambda i: (i, 0))],
        out_specs=pl.BlockSpec((BM, d), lambda i: (i, 0)),
        out_shape=jax.ShapeDtypeStruct((n, d), jnp.float32),
    )(q, stats, g2, be2, x)

    return h
